# baseline (speedup 1.0000x reference)
import functools

import jax
import jax.numpy as jnp
from jax import lax
from jax.experimental import pallas as pl
from jax.experimental.pallas import tpu as pltpu
from jax.experimental.pallas import tpu_sc as plsc

_N = 10000
_D = 128
_E = 320000

_NC = 2
_NS = 16
_NW = _NC * _NS
_K = 128
_CH = 80
_NP = 2
_CHP = _CH // _NP
_EPW = _CH * _K
_EPAD = _NW * _EPW
_NPAD = 10240
_TROWS = _NPAD // _NS

_mesh = plsc.VectorSubcoreMesh(core_axis_name="c", subcore_axis_name="s")


def _zero_vmem_2d(buf, rows, cols):
    zv = jnp.zeros((16,), jnp.float32)

    def _row(i, _):
        def _col(j, _):
            buf[i, pl.ds(j * 16, 16)] = zv
            return 0
        return lax.fori_loop(0, cols // 16, _col, 0)

    lax.fori_loop(0, rows, _row, 0)


def _fill_vmem_1d(buf, n, value):
    vv = jnp.full((16,), value, jnp.float32)

    def _b(i, _):
        buf[pl.ds(i * 16, 16)] = vv
        return 0

    lax.fori_loop(0, n // 16, _b, 0)


@functools.partial(
    pl.kernel,
    out_type=[
        jax.ShapeDtypeStruct((_NC, _NPAD), jnp.float32),
        jax.ShapeDtypeStruct((_NC, _NPAD), jnp.float32),
    ],
    mesh=_mesh,
    scratch_types=[
        pltpu.VMEM((_EPW,), jnp.int32),
        pltpu.VMEM((_EPW,), jnp.int32),
        pltpu.VMEM((_K,), jnp.float32),
        pltpu.VMEM((_K,), jnp.float32),
        pltpu.VMEM_SHARED((_NPAD,), jnp.float32),
        pltpu.VMEM_SHARED((_NPAD,), jnp.float32),
    ],
)
def _deg_kernel(src_hbm, dst_hbm, dout_hbm, din_hbm,
                src_v, dst_v, ones_v, zeros_v, dout_sh, din_sh):
    c = lax.axis_index("c")
    s = lax.axis_index("s")
    wid = c * _NS + s

    pltpu.sync_copy(src_hbm.at[wid, pl.ds(0, _EPW)], src_v)
    pltpu.sync_copy(dst_hbm.at[wid, pl.ds(0, _EPW)], dst_v)

    _fill_vmem_1d(ones_v, _K, 1.0)
    _fill_vmem_1d(zeros_v, _K, 0.0)

    def _zs(t, _):
        off = s * _TROWS + t * _K
        pltpu.sync_copy(zeros_v, dout_sh.at[pl.ds(off, _K)])
        pltpu.sync_copy(zeros_v, din_sh.at[pl.ds(off, _K)])
        return 0

    lax.fori_loop(0, _TROWS // _K, _zs, 0)
    plsc.subcore_barrier()

    def _acc(j, _):
        pltpu.sync_copy(ones_v, dout_sh.at[src_v.at[pl.ds(j * _K, _K)]],
                        add=True)
        pltpu.sync_copy(ones_v, din_sh.at[dst_v.at[pl.ds(j * _K, _K)]],
                        add=True)
        return 0

    lax.fori_loop(0, _EPW // _K, _acc, 0)
    plsc.subcore_barrier()

    off = s * _TROWS
    pltpu.sync_copy(dout_sh.at[pl.ds(off, _TROWS)],
                    dout_hbm.at[c, pl.ds(off, _TROWS)])
    pltpu.sync_copy(din_sh.at[pl.ds(off, _TROWS)],
                    din_hbm.at[c, pl.ds(off, _TROWS)])


_K2 = 64
_NCH = _CHP * _K // _K2
_IDXW = _CHP * _K


@functools.partial(
    pl.kernel,
    out_type=jax.ShapeDtypeStruct((_NC, _NPAD, _D), jnp.float32),
    mesh=_mesh,
    scratch_types=[
        pltpu.VMEM((_IDXW,), jnp.int32),
        pltpu.VMEM((_IDXW,), jnp.int32),
        pltpu.VMEM((_K2, _D), jnp.float32),
        pltpu.VMEM((_K2, _D), jnp.float32),
        pltpu.VMEM((_K2, _D), jnp.float32),
        pltpu.VMEM((_K2, _D), jnp.float32),
        pltpu.VMEM_SHARED((_NPAD, _D), jnp.float32),
        pltpu.SemaphoreType.DMA,
        pltpu.SemaphoreType.DMA,
        pltpu.SemaphoreType.DMA,
        pltpu.SemaphoreType.DMA,
        pltpu.SemaphoreType.DMA,
        pltpu.SemaphoreType.DMA,
        pltpu.SemaphoreType.DMA,
        pltpu.SemaphoreType.DMA,
    ],
)
def _agg_kernel(y_hbm, src_hbm, dst_hbm, out_hbm,
                src_v, dst_v, r0, r1, r2, r3, agg_sh,
                g0, g1, g2, g3, s0, s1, s2, s3):
    c = lax.axis_index("c")
    s = lax.axis_index("s")
    wid = c * _NS + s
    rows = [r0, r1, r2, r3]
    gsem = [g0, g1, g2, g3]
    ssem = [s0, s1, s2, s3]

    def _sidx(j):
        return src_v.at[pl.ds(j * _K2, _K2)]

    def _didx(j):
        return dst_v.at[pl.ds(j * _K2, _K2)]

    def _gather(j, b):
        pltpu.async_copy(y_hbm.at[_sidx(j)], rows[b], gsem[b])

    def _gwait(j, b):
        pltpu.make_async_copy(y_hbm.at[_sidx(j)], rows[b], gsem[b]).wait()

    def _scat(j, b):
        pltpu.async_copy(rows[b], agg_sh.at[_didx(j)], ssem[b], add=True)

    def _swait(j, b):
        pltpu.make_async_copy(rows[b], agg_sh.at[_didx(j)], ssem[b]).wait()

    _zero_vmem_2d(r0, _K2, _D)

    def _zs(t, _):
        pltpu.sync_copy(r0, agg_sh.at[pl.ds(s * _TROWS + t * _K2, _K2)])
        return 0

    lax.fori_loop(0, _TROWS // _K2, _zs, 0)
    plsc.subcore_barrier()

    def _pass(p, _):
        pltpu.sync_copy(src_hbm.at[wid, pl.ds(p * _IDXW, _IDXW)], src_v)
        pltpu.sync_copy(dst_hbm.at[wid, pl.ds(p * _IDXW, _IDXW)], dst_v)

        _gather(0, 0)
        _gather(1, 1)
        _gather(2, 2)
        _gwait(0, 0)
        _scat(0, 0)
        _gather(3, 3)
        _gwait(1, 1)
        _scat(1, 1)

        def _step(g, _):
            for i in range(4):
                j = 2 + 4 * g + i
                bn = i
                b = (2 + i) % 4
                _swait(j - 2, bn)
                _gather(j + 2, bn)
                _gwait(j, b)
                _scat(j, b)
            return 0

        lax.fori_loop(0, (_NCH - 4) // 4, _step, 0)

        _gwait(_NCH - 2, (_NCH - 2) % 4)
        _scat(_NCH - 2, (_NCH - 2) % 4)
        _gwait(_NCH - 1, (_NCH - 1) % 4)
        _scat(_NCH - 1, (_NCH - 1) % 4)
        _swait(_NCH - 4, 0)
        _swait(_NCH - 3, 1)
        _swait(_NCH - 2, 2)
        _swait(_NCH - 1, 3)
        return 0

    lax.fori_loop(0, _NP, _pass, 0)

    plsc.subcore_barrier()
    pltpu.sync_copy(agg_sh.at[pl.ds(s * _TROWS, _TROWS)],
                    out_hbm.at[c, pl.ds(s * _TROWS, _TROWS)])


def _mm_body(x_ref, w_ref, o_ref):
    o_ref[...] = jnp.dot(x_ref[...], w_ref[...],
                         preferred_element_type=jnp.float32)


_mm = pl.pallas_call(
    _mm_body,
    out_shape=jax.ShapeDtypeStruct((_NPAD, _D), jnp.float32),
)


def _norms_body(dout_ref, din_ref, xw_ref, y_ref, ns_ref, nd_ref):
    dout = dout_ref[0] + dout_ref[1]
    din = din_ref[0] + din_ref[1]
    ns = jnp.where(dout > 0, lax.rsqrt(jnp.maximum(dout, 1e-12)), 0.0)
    nd = jnp.where(din > 0, lax.rsqrt(jnp.maximum(din, 1e-12)), 0.0)
    ns_ref[...] = ns
    nd_ref[...] = nd
    y_ref[...] = xw_ref[...] * ns


_norms = pl.pallas_call(
    _norms_body,
    out_shape=[
        jax.ShapeDtypeStruct((_NPAD, _D), jnp.float32),
        jax.ShapeDtypeStruct((_NPAD, 1), jnp.float32),
        jax.ShapeDtypeStruct((_NPAD, 1), jnp.float32),
    ],
)


def _layer_body(z_ref, nd_ref, b_ref, w_ref, ns_ref, y_ref):
    agg = (z_ref[0] + z_ref[1]) * nd_ref[...]
    h = jnp.maximum(agg + b_ref[...], 0.0)
    y_ref[...] = jnp.dot(h, w_ref[...],
                         preferred_element_type=jnp.float32) * ns_ref[...]


_layer = pl.pallas_call(
    _layer_body,
    out_shape=jax.ShapeDtypeStruct((_NPAD, _D), jnp.float32),
)


def _final_body(z_ref, nd_ref, b_ref, r1w_ref, r1b_ref, r2w_ref, r2b_ref,
                r_ref, h_ref):
    agg = (z_ref[0] + z_ref[1]) * nd_ref[...]
    h = jnp.maximum(agg + b_ref[...], 0.0)
    h_ref[...] = h
    row = lax.broadcasted_iota(jnp.int32, (_NPAD, 1), 0)
    hm = jnp.where(row < _N, h, 0.0)
    hg = jnp.sum(hm, axis=0, keepdims=True) * (1.0 / _N)
    t = jnp.maximum(
        jnp.dot(hg, r1w_ref[...], preferred_element_type=jnp.float32)
        + r1b_ref[...], 0.0)
    r_ref[...] = (jnp.dot(t, r2w_ref[...], preferred_element_type=jnp.float32)
                  + r2b_ref[...])


_final = pl.pallas_call(
    _final_body,
    out_shape=[
        jax.ShapeDtypeStruct((1, _D), jnp.float32),
        jax.ShapeDtypeStruct((_NPAD, _D), jnp.float32),
    ],
)


def kernel(x, edge_index, W0, b0, W1, b1, R1_w, R1_b, R2_w, R2_b):
    src = edge_index[0]
    dst = edge_index[1]
    padlen = _EPAD - _E
    fill = (_N + (jnp.arange(padlen, dtype=jnp.int32) % (_NPAD - _N)))
    src2 = jnp.concatenate([src, fill]).reshape(_NW, _EPW)
    dst2 = jnp.concatenate([dst, fill]).reshape(_NW, _EPW)
    x_p = jnp.zeros((_NPAD, _D), jnp.float32).at[:_N].set(x)

    xw0 = _mm(x_p, W0)
    dout, din = _deg_kernel(src2, dst2)
    y0, ns, nd = _norms(dout.reshape(_NC, _NPAD, 1),
                        din.reshape(_NC, _NPAD, 1), xw0)
    z0 = _agg_kernel(y0, src2, dst2)
    y1 = _layer(z0, nd, b0.reshape(1, _D), W1, ns)
    z1 = _agg_kernel(y1, src2, dst2)
    r, h = _final(z1, nd, b1.reshape(1, _D), R1_w, R1_b.reshape(1, _D),
                  R2_w, R2_b.reshape(1, _D))
    return (r, h[:_N])


# speedup vs baseline: 1.0158x; 1.0158x over previous
import functools

import jax
import jax.numpy as jnp
from jax import lax
from jax.experimental import pallas as pl
from jax.experimental.pallas import tpu as pltpu
from jax.experimental.pallas import tpu_sc as plsc

_N = 10000
_D = 128
_E = 320000

_NC = 2
_NS = 16
_NW = _NC * _NS
_K = 128
_CH = 80
_NP = 2
_CHP = _CH // _NP
_EPW = _CH * _K
_EPAD = _NW * _EPW
_NPAD = 10240
_TROWS = _NPAD // _NS

_mesh = plsc.VectorSubcoreMesh(core_axis_name="c", subcore_axis_name="s")


def _zero_vmem_2d(buf, rows, cols):
    zv = jnp.zeros((16,), jnp.float32)

    def _row(i, _):
        def _col(j, _):
            buf[i, pl.ds(j * 16, 16)] = zv
            return 0
        return lax.fori_loop(0, cols // 16, _col, 0)

    lax.fori_loop(0, rows, _row, 0)


def _fill_vmem_1d(buf, n, value):
    vv = jnp.full((16,), value, jnp.float32)

    def _b(i, _):
        buf[pl.ds(i * 16, 16)] = vv
        return 0

    lax.fori_loop(0, n // 16, _b, 0)


@functools.partial(
    pl.kernel,
    out_type=[
        jax.ShapeDtypeStruct((_NC, _NPAD), jnp.float32),
        jax.ShapeDtypeStruct((_NC, _NPAD), jnp.float32),
    ],
    mesh=_mesh,
    scratch_types=[
        pltpu.VMEM((_EPW,), jnp.int32),
        pltpu.VMEM((_EPW,), jnp.int32),
        pltpu.VMEM((_K,), jnp.float32),
        pltpu.VMEM((_K,), jnp.float32),
        pltpu.VMEM_SHARED((_NPAD,), jnp.float32),
        pltpu.VMEM_SHARED((_NPAD,), jnp.float32),
    ],
)
def _deg_kernel(src_hbm, dst_hbm, dout_hbm, din_hbm,
                src_v, dst_v, ones_v, zeros_v, dout_sh, din_sh):
    c = lax.axis_index("c")
    s = lax.axis_index("s")
    wid = c * _NS + s

    pltpu.sync_copy(src_hbm.at[wid, pl.ds(0, _EPW)], src_v)
    pltpu.sync_copy(dst_hbm.at[wid, pl.ds(0, _EPW)], dst_v)

    _fill_vmem_1d(ones_v, _K, 1.0)
    _fill_vmem_1d(zeros_v, _K, 0.0)

    def _zs(t, _):
        off = s * _TROWS + t * _K
        pltpu.sync_copy(zeros_v, dout_sh.at[pl.ds(off, _K)])
        pltpu.sync_copy(zeros_v, din_sh.at[pl.ds(off, _K)])
        return 0

    lax.fori_loop(0, _TROWS // _K, _zs, 0)
    plsc.subcore_barrier()

    def _acc(j, _):
        pltpu.sync_copy(ones_v, dout_sh.at[src_v.at[pl.ds(j * _K, _K)]],
                        add=True)
        pltpu.sync_copy(ones_v, din_sh.at[dst_v.at[pl.ds(j * _K, _K)]],
                        add=True)
        return 0

    lax.fori_loop(0, _EPW // _K, _acc, 0)
    plsc.subcore_barrier()

    off = s * _TROWS
    pltpu.sync_copy(dout_sh.at[pl.ds(off, _TROWS)],
                    dout_hbm.at[c, pl.ds(off, _TROWS)])
    pltpu.sync_copy(din_sh.at[pl.ds(off, _TROWS)],
                    din_hbm.at[c, pl.ds(off, _TROWS)])


_K2 = 64
_NCH = _CHP * _K // _K2
_IDXW = _CHP * _K


@functools.partial(
    pl.kernel,
    out_type=jax.ShapeDtypeStruct((_NC, _NPAD, _D), jnp.float32),
    mesh=_mesh,
    scratch_types=[
        pltpu.VMEM((_IDXW,), jnp.int32),
        pltpu.VMEM((_IDXW,), jnp.int32),
        pltpu.VMEM((_K2, _D), jnp.float32),
        pltpu.VMEM((_K2, _D), jnp.float32),
        pltpu.VMEM((_K2, _D), jnp.float32),
        pltpu.VMEM((_K2, _D), jnp.float32),
        pltpu.VMEM_SHARED((_NPAD, _D), jnp.float32),
        pltpu.SemaphoreType.DMA,
        pltpu.SemaphoreType.DMA,
        pltpu.SemaphoreType.DMA,
        pltpu.SemaphoreType.DMA,
        pltpu.SemaphoreType.DMA,
        pltpu.SemaphoreType.DMA,
        pltpu.SemaphoreType.DMA,
        pltpu.SemaphoreType.DMA,
    ],
)
def _agg_kernel(y_hbm, src_hbm, dst_hbm, out_hbm,
                src_v, dst_v, r0, r1, r2, r3, agg_sh,
                g0, g1, g2, g3, s0, s1, s2, s3):
    c = lax.axis_index("c")
    s = lax.axis_index("s")
    wid = c * _NS + s
    rows = [r0, r1, r2, r3]
    gsem = [g0, g1, g2, g3]
    ssem = [s0, s1, s2, s3]

    def _sidx(j):
        return src_v.at[pl.ds(j * _K2, _K2)]

    def _didx(j):
        return dst_v.at[pl.ds(j * _K2, _K2)]

    def _gather(j, b):
        pltpu.async_copy(y_hbm.at[_sidx(j)], rows[b], gsem[b])

    def _gwait(j, b):
        pltpu.make_async_copy(y_hbm.at[_sidx(j)], rows[b], gsem[b]).wait()

    def _scat(j, b):
        pltpu.async_copy(rows[b], agg_sh.at[_didx(j)], ssem[b], add=True)

    def _swait(j, b):
        pltpu.make_async_copy(rows[b], agg_sh.at[_didx(j)], ssem[b]).wait()

    _zero_vmem_2d(r0, _K2, _D)

    def _zs(t, _):
        pltpu.sync_copy(r0, agg_sh.at[pl.ds(s * _TROWS + t * _K2, _K2)])
        return 0

    lax.fori_loop(0, _TROWS // _K2, _zs, 0)
    plsc.subcore_barrier()

    def _pass(p, _):
        pltpu.sync_copy(src_hbm.at[wid, pl.ds(p * _IDXW, _IDXW)], src_v)
        pltpu.sync_copy(dst_hbm.at[wid, pl.ds(p * _IDXW, _IDXW)], dst_v)

        _gather(0, 0)
        _gather(1, 1)
        _gather(2, 2)
        _gwait(0, 0)
        _scat(0, 0)
        _gather(3, 3)
        _gwait(1, 1)
        _scat(1, 1)

        def _step(g, _):
            for i in range(4):
                j = 2 + 4 * g + i
                bn = i
                b = (2 + i) % 4
                _swait(j - 2, bn)
                _gather(j + 2, bn)
                _gwait(j, b)
                _scat(j, b)
            return 0

        lax.fori_loop(0, (_NCH - 4) // 4, _step, 0)

        _gwait(_NCH - 2, (_NCH - 2) % 4)
        _scat(_NCH - 2, (_NCH - 2) % 4)
        _gwait(_NCH - 1, (_NCH - 1) % 4)
        _scat(_NCH - 1, (_NCH - 1) % 4)
        _swait(_NCH - 4, 0)
        _swait(_NCH - 3, 1)
        _swait(_NCH - 2, 2)
        _swait(_NCH - 1, 3)
        return 0

    lax.fori_loop(0, _NP, _pass, 0)

    plsc.subcore_barrier()
    pltpu.sync_copy(agg_sh.at[pl.ds(s * _TROWS, _TROWS)],
                    out_hbm.at[c, pl.ds(s * _TROWS, _TROWS)])


def _norms_body(dout_ref, din_ref, x_ref, w_ref, y_ref, ns_ref, nd_ref):
    dout = dout_ref[0] + dout_ref[1]
    din = din_ref[0] + din_ref[1]
    ns = jnp.where(dout > 0, lax.rsqrt(jnp.maximum(dout, 1e-12)), 0.0)
    nd = jnp.where(din > 0, lax.rsqrt(jnp.maximum(din, 1e-12)), 0.0)
    ns_ref[...] = ns
    nd_ref[...] = nd
    xw = jnp.dot(x_ref[...], w_ref[...], preferred_element_type=jnp.float32)
    y_ref[0:_N, :] = xw * ns[0:_N]
    y_ref[_N:_NPAD, :] = jnp.zeros((_NPAD - _N, _D), jnp.float32)


_norms = pl.pallas_call(
    _norms_body,
    out_shape=[
        jax.ShapeDtypeStruct((_NPAD, _D), jnp.float32),
        jax.ShapeDtypeStruct((_NPAD, 1), jnp.float32),
        jax.ShapeDtypeStruct((_NPAD, 1), jnp.float32),
    ],
)


def _layer_body(z_ref, nd_ref, b_ref, w_ref, ns_ref, y_ref):
    agg = (z_ref[0] + z_ref[1]) * nd_ref[...]
    h = jnp.maximum(agg + b_ref[...], 0.0)
    y_ref[...] = jnp.dot(h, w_ref[...],
                         preferred_element_type=jnp.float32) * ns_ref[...]


_layer = pl.pallas_call(
    _layer_body,
    out_shape=jax.ShapeDtypeStruct((_NPAD, _D), jnp.float32),
)


def _final_body(z_ref, nd_ref, b_ref, r1w_ref, r1b_ref, r2w_ref, r2b_ref,
                r_ref, h_ref):
    agg = (z_ref[0, 0:_N] + z_ref[1, 0:_N]) * nd_ref[0:_N]
    h = jnp.maximum(agg + b_ref[...], 0.0)
    h_ref[...] = h
    hg = jnp.sum(h, axis=0, keepdims=True) * (1.0 / _N)
    t = jnp.maximum(
        jnp.dot(hg, r1w_ref[...], preferred_element_type=jnp.float32)
        + r1b_ref[...], 0.0)
    r_ref[...] = (jnp.dot(t, r2w_ref[...], preferred_element_type=jnp.float32)
                  + r2b_ref[...])


_final = pl.pallas_call(
    _final_body,
    out_shape=[
        jax.ShapeDtypeStruct((1, _D), jnp.float32),
        jax.ShapeDtypeStruct((_N, _D), jnp.float32),
    ],
)


def kernel(x, edge_index, W0, b0, W1, b1, R1_w, R1_b, R2_w, R2_b):
    src = edge_index[0]
    dst = edge_index[1]
    padlen = _EPAD - _E
    fill = (_N + (jnp.arange(padlen, dtype=jnp.int32) % (_NPAD - _N)))
    src2 = jnp.concatenate([src, fill]).reshape(_NW, _EPW)
    dst2 = jnp.concatenate([dst, fill]).reshape(_NW, _EPW)

    dout, din = _deg_kernel(src2, dst2)
    y0, ns, nd = _norms(dout.reshape(_NC, _NPAD, 1),
                        din.reshape(_NC, _NPAD, 1), x, W0)
    z0 = _agg_kernel(y0, src2, dst2)
    y1 = _layer(z0, nd, b0.reshape(1, _D), W1, ns)
    z1 = _agg_kernel(y1, src2, dst2)
    r, h = _final(z1, nd, b1.reshape(1, _D), R1_w, R1_b.reshape(1, _D),
                  R2_w, R2_b.reshape(1, _D))
    return (r, h)
